# baseline (speedup 1.0000x reference)
import functools

import jax
import jax.numpy as jnp
from jax import lax
from jax.experimental import pallas as pl
from jax.experimental.pallas import tpu as pltpu
from jax.experimental.pallas import tpu_sc as plsc

_B = 8
_N = 20000
_F = 85
_C = 80
_ROWS = _B * _N
_THRES = 0.05

_NC = 2
_NS = 16
_NW = _NC * _NS

_CHUNK = 128
_GROUPS = _CHUNK // 16
_PER_W = (_ROWS // _NW) // _CHUNK * _CHUNK
_NCHUNK = _PER_W // _CHUNK
_TAIL_ROW0 = _NW * _PER_W
_TAIL_CHUNKS = (_ROWS - _TAIL_ROW0) // _CHUNK


def _process_chunk(row0, pred_hbm, det_hbm, mask_hbm, in_v, det_v, mask_v,
                   riota85, riota7):
    pltpu.sync_copy(pred_hbm.at[pl.ds(row0 * _F, _CHUNK * _F)], in_v)

    def group_body(g, carry):
        base = riota85 + g * (16 * _F)
        cols = [plsc.load_gather(in_v, [base + c]) for c in range(4)]
        conf = plsc.load_gather(in_v, [base + 4])
        maxv = plsc.load_gather(in_v, [base + 5])
        amax = jnp.zeros((16,), jnp.float32)
        for c in range(1, _C):
            v = plsc.load_gather(in_v, [base + (5 + c)])
            upd = v > maxv
            maxv = jnp.where(upd, v, maxv)
            amax = jnp.where(upd, jnp.float32(c), amax)
        mask = conf >= _THRES
        didx = riota7 + g * (16 * 7)
        for c, val in enumerate(cols + [conf, maxv, amax]):
            plsc.store_scatter(det_v, [didx + c], jnp.where(mask, val, 0.0))
        mask_v[pl.ds(g * 16, 16)] = mask.astype(jnp.int32)
        return carry

    lax.fori_loop(0, _GROUPS, group_body, 0)
    pltpu.sync_copy(det_v, det_hbm.at[pl.ds(row0 * 7, _CHUNK * 7)])
    pltpu.sync_copy(mask_v, mask_hbm.at[pl.ds(row0, _CHUNK)])


@functools.partial(
    pl.kernel,
    mesh=plsc.VectorSubcoreMesh(core_axis_name="c", subcore_axis_name="s"),
    out_type=(
        jax.ShapeDtypeStruct((_ROWS * 7,), jnp.float32),
        jax.ShapeDtypeStruct((_ROWS,), jnp.int32),
    ),
    scratch_types=[
        pltpu.VMEM((_CHUNK * _F,), jnp.float32),
        pltpu.VMEM((_CHUNK * 7,), jnp.float32),
        pltpu.VMEM((_CHUNK,), jnp.int32),
    ],
    compiler_params=pltpu.CompilerParams(needs_layout_passes=False),
)
def _sc_postprocess(pred_hbm, det_hbm, mask_hbm, in_v, det_v, mask_v):
    wid = lax.axis_index("s") * _NC + lax.axis_index("c")
    start = wid * _PER_W
    riota85 = lax.iota(jnp.int32, 16) * _F
    riota7 = lax.iota(jnp.int32, 16) * 7

    def chunk_body(i, carry):
        _process_chunk(start + i * _CHUNK, pred_hbm, det_hbm, mask_hbm,
                       in_v, det_v, mask_v, riota85, riota7)
        return carry

    lax.fori_loop(0, _NCHUNK, chunk_body, 0)

    @pl.when(wid < _TAIL_CHUNKS)
    def _tail():
        _process_chunk(_TAIL_ROW0 + wid * _CHUNK, pred_hbm, det_hbm,
                       mask_hbm, in_v, det_v, mask_v, riota85, riota7)


def kernel(prediction):
    det, mask = _sc_postprocess(prediction.reshape(-1))
    return (det.reshape(_B, _N, 7),
            mask.reshape(_B, _N).astype(jnp.bool_))


# speedup vs baseline: 1.2160x; 1.2160x over previous
import functools

import jax
import jax.numpy as jnp
from jax import lax
from jax.experimental import pallas as pl
from jax.experimental.pallas import tpu as pltpu
from jax.experimental.pallas import tpu_sc as plsc

_B = 8
_N = 20000
_F = 85
_C = 80
_ROWS = _B * _N
_THRES = 0.05

_NC = 2
_NS = 16
_NW = _NC * _NS

_CHUNK = 128
_GROUPS = _CHUNK // 16
_NCHUNKS = _ROWS // _CHUNK
_PER_W = 40
_LAST_START = _NCHUNKS - _PER_W


def _merge(lv, li, rv, ri):
    upd = rv > lv
    return jnp.where(upd, rv, lv), jnp.where(upd, ri, li)


def _gather_col(in_ref, base_r, c):
    sub = (c // 8) * 8
    return plsc.load_gather(in_ref.at[pl.ds(sub, _CHUNK * _F - 80)],
                            [base_r[c % 8]])


def _class_argmax(in_ref, base_r):
    waves = []
    for w in range(5):
        c0 = 5 + 16 * w
        vs = [_gather_col(in_ref, base_r, c0 + j) for j in range(16)]
        lvl = []
        for j in range(8):
            upd = vs[2 * j + 1] > vs[2 * j]
            lvl.append((jnp.where(upd, vs[2 * j + 1], vs[2 * j]),
                        jnp.where(upd, jnp.float32(16 * w + 2 * j + 1),
                                  jnp.float32(16 * w + 2 * j))))
        while len(lvl) > 1:
            lvl = [_merge(*lvl[2 * k], *lvl[2 * k + 1])
                   for k in range(len(lvl) // 2)]
        waves.append(lvl[0])
    mv, mi = _merge(*_merge(*waves[0], *waves[1]), *_merge(*waves[2], *waves[3]))
    return _merge(mv, mi, *waves[4])


def _compute_chunk(in_ref, det_ref, mask_ref, riota85, riota7):
    @plsc.parallel_loop(0, _GROUPS, unroll=1)
    def _(g):
        base = riota85 + g * (16 * _F)
        base_r = [base + r if r else base for r in range(8)]
        cols = [_gather_col(in_ref, base_r, c) for c in range(5)]
        maxv, amax = _class_argmax(in_ref, base_r)
        mask = cols[4] >= _THRES
        didx = riota7 + g * (16 * 7)
        for c, val in enumerate(cols + [maxv, amax]):
            plsc.store_scatter(det_ref, [didx + c if c else didx],
                               jnp.where(mask, val, 0.0))
        mask_ref[pl.ds(g * 16, 16)] = mask.astype(jnp.int32)


@functools.partial(
    pl.kernel,
    mesh=plsc.VectorSubcoreMesh(core_axis_name="c", subcore_axis_name="s"),
    out_type=(
        jax.ShapeDtypeStruct((_ROWS * 7,), jnp.float32),
        jax.ShapeDtypeStruct((_ROWS,), jnp.int32),
    ),
    scratch_types=[
        pltpu.VMEM((_CHUNK * _F,), jnp.float32),
        pltpu.VMEM((_CHUNK * _F,), jnp.float32),
        pltpu.VMEM((_CHUNK * 7,), jnp.float32),
        pltpu.VMEM((_CHUNK * 7,), jnp.float32),
        pltpu.VMEM((_CHUNK,), jnp.int32),
        pltpu.VMEM((_CHUNK,), jnp.int32),
        pltpu.SemaphoreType.DMA,
        pltpu.SemaphoreType.DMA,
        pltpu.SemaphoreType.DMA,
        pltpu.SemaphoreType.DMA,
    ],
    compiler_params=pltpu.CompilerParams(needs_layout_passes=False),
)
def _sc_postprocess(pred_hbm, det_hbm, mask_hbm,
                    in_v0, in_v1, det_v0, det_v1, mask_v0, mask_v1,
                    in_s0, in_s1, out_s0, out_s1):
    wid = lax.axis_index("s") * _NC + lax.axis_index("c")
    start = (wid * _LAST_START) // (_NW - 1)
    riota85 = lax.iota(jnp.int32, 16) * _F
    riota7 = lax.iota(jnp.int32, 16) * 7

    in_v = (in_v0, in_v1)
    det_v = (det_v0, det_v1)
    mask_v = (mask_v0, mask_v1)
    in_s = (in_s0, in_s1)
    out_s = (out_s0, out_s1)

    def in_copy(i, b):
        row0 = (start + i) * _CHUNK
        return pltpu.make_async_copy(
            pred_hbm.at[pl.ds(row0 * _F, _CHUNK * _F)], in_v[b], in_s[b])

    def det_copy(i, b):
        row0 = (start + i) * _CHUNK
        return pltpu.make_async_copy(
            det_v[b], det_hbm.at[pl.ds(row0 * 7, _CHUNK * 7)], out_s[b])

    def mask_copy(i, b):
        row0 = (start + i) * _CHUNK
        return pltpu.make_async_copy(
            mask_v[b], mask_hbm.at[pl.ds(row0, _CHUNK)], out_s[b])

    def chunk_body(i, b):
        in_copy(i, b).wait()

        @pl.when(i >= 2)
        def _drain_out():
            det_copy(i, b).wait()
            mask_copy(i, b).wait()

        _compute_chunk(in_v[b], det_v[b], mask_v[b], riota85, riota7)
        det_copy(i, b).start()
        mask_copy(i, b).start()

        @pl.when(i + 2 < _PER_W)
        def _prefetch():
            in_copy(i + 2, b).start()

    in_copy(0, 0).start()
    in_copy(1, 1).start()

    def pair_body(j, carry):
        chunk_body(2 * j, 0)
        chunk_body(2 * j + 1, 1)
        return carry

    lax.fori_loop(0, _PER_W // 2, pair_body, 0)

    for b in range(2):
        det_copy(_PER_W - 2 + b, b).wait()
        mask_copy(_PER_W - 2 + b, b).wait()


def kernel(prediction):
    det, mask = _sc_postprocess(prediction.reshape(-1))
    return (det.reshape(_B, _N, 7),
            mask.reshape(_B, _N).astype(jnp.bool_))


# speedup vs baseline: 1.2622x; 1.0380x over previous
import jax
import jax.numpy as jnp
from jax import lax
from jax.experimental import pallas as pl

_B = 8
_N = 20000
_F = 85
_C = 80
_ROWS = _B * _N
_THRES = 0.05

_BLK = 2000
_NB = _ROWS // _BLK


def _postprocess_block(in_ref, det_ref, mask_ref):
    x = in_ref[...]
    obj = x[:, 4:5]
    cls = x[:, 5:5 + _C]
    cmax = jnp.max(cls, axis=1, keepdims=True)
    lane = lax.broadcasted_iota(jnp.int32, cls.shape, 1)
    carg = jnp.min(jnp.where(cls == cmax, lane, _C), axis=1, keepdims=True)
    mask = obj >= _THRES
    det = jnp.concatenate([x[:, 0:4], obj, cmax, carg.astype(jnp.float32)],
                          axis=1)
    det_ref[...] = jnp.where(mask, det, 0.0)
    mask_ref[...] = mask


def kernel(prediction):
    det, mask = pl.pallas_call(
        _postprocess_block,
        grid=(_NB,),
        in_specs=[pl.BlockSpec((_BLK, _F), lambda i: (i, 0))],
        out_specs=[
            pl.BlockSpec((_BLK, 7), lambda i: (i, 0)),
            pl.BlockSpec((_BLK, 1), lambda i: (i, 0)),
        ],
        out_shape=[
            jax.ShapeDtypeStruct((_ROWS, 7), jnp.float32),
            jax.ShapeDtypeStruct((_ROWS, 1), jnp.bool_),
        ],
    )(prediction.reshape(_ROWS, _F))
    return det.reshape(_B, _N, 7), mask.reshape(_B, _N)


# speedup vs baseline: 24.2859x; 19.2406x over previous
import jax
import jax.numpy as jnp
from jax.experimental import pallas as pl

_B = 8
_N = 20000
_F = 85
_C = 80
_THRES = 0.05

_BLKN = 2048
_NB = -(-_N // _BLKN)


def _postprocess_block(in_ref, det_ref, mask_ref):
    obj = in_ref[4]
    mask = obj >= _THRES
    best = in_ref[5]
    bidx = jnp.zeros_like(best)
    for c in range(1, _C):
        v = in_ref[5 + c]
        upd = v > best
        best = jnp.where(upd, v, best)
        bidx = jnp.where(upd, jnp.float32(c), bidx)
    zero = jnp.zeros_like(best)
    for c in range(4):
        det_ref[c] = jnp.where(mask, in_ref[c], zero)
    det_ref[4] = jnp.where(mask, obj, zero)
    det_ref[5] = jnp.where(mask, best, zero)
    det_ref[6] = jnp.where(mask, bidx, zero)
    mask_ref[...] = mask


def kernel(prediction):
    xp = jnp.transpose(prediction, (2, 0, 1))
    det_p, mask = pl.pallas_call(
        _postprocess_block,
        grid=(_NB,),
        in_specs=[pl.BlockSpec((_F, _B, _BLKN), lambda i: (0, 0, i))],
        out_specs=[
            pl.BlockSpec((7, _B, _BLKN), lambda i: (0, 0, i)),
            pl.BlockSpec((_B, _BLKN), lambda i: (0, i)),
        ],
        out_shape=[
            jax.ShapeDtypeStruct((7, _B, _N), jnp.float32),
            jax.ShapeDtypeStruct((_B, _N), jnp.bool_),
        ],
    )(xp)
    return jnp.transpose(det_p, (1, 2, 0)), mask


# speedup vs baseline: 24.9506x; 1.0274x over previous
import jax
import jax.numpy as jnp
from jax.experimental import pallas as pl

_B = 8
_N = 20000
_F = 85
_C = 80
_THRES = 0.05

_BLKN = 4096
_NB = -(-_N // _BLKN)


def _postprocess_block(in_ref, det_ref, mask_ref):
    obj = in_ref[4]
    mask = obj >= _THRES
    best = in_ref[5]
    bidx = jnp.zeros_like(best)
    for c in range(1, _C):
        v = in_ref[5 + c]
        upd = v > best
        best = jnp.where(upd, v, best)
        bidx = jnp.where(upd, jnp.float32(c), bidx)
    zero = jnp.zeros_like(best)
    for c in range(4):
        det_ref[c] = jnp.where(mask, in_ref[c], zero)
    det_ref[4] = jnp.where(mask, obj, zero)
    det_ref[5] = jnp.where(mask, best, zero)
    det_ref[6] = jnp.where(mask, bidx, zero)
    mask_ref[...] = mask


def kernel(prediction):
    xp = jnp.transpose(prediction, (2, 0, 1))
    det_p, mask = pl.pallas_call(
        _postprocess_block,
        grid=(_NB,),
        in_specs=[pl.BlockSpec((_F, _B, _BLKN), lambda i: (0, 0, i))],
        out_specs=[
            pl.BlockSpec((7, _B, _BLKN), lambda i: (0, 0, i)),
            pl.BlockSpec((_B, _BLKN), lambda i: (0, i)),
        ],
        out_shape=[
            jax.ShapeDtypeStruct((7, _B, _N), jnp.float32),
            jax.ShapeDtypeStruct((_B, _N), jnp.bool_),
        ],
    )(xp)
    return jnp.transpose(det_p, (1, 2, 0)), mask
